# baseline (speedup 1.0000x reference)
import functools

import jax
import jax.numpy as jnp
from jax import lax
from jax.experimental import pallas as pl
from jax.experimental.pallas import tpu as pltpu
from jax.experimental.pallas import tpu_sc as plsc

B = 4096
D = 64

_info = plsc.get_sparse_core_info()
_NC = _info.num_cores
_NS = _info.num_subcores
_NW = _NC * _NS
_BPW = B // _NW


@functools.partial(
    pl.kernel,
    mesh=plsc.VectorSubcoreMesh(core_axis_name="c", subcore_axis_name="s"),
    out_type=[
        jax.ShapeDtypeStruct((B, D), jnp.float32),
        jax.ShapeDtypeStruct((B, D), jnp.float32),
        jax.ShapeDtypeStruct((B,), jnp.float32),
        jax.ShapeDtypeStruct((B,), jnp.float32),
    ],
    scratch_types=[
        pltpu.VMEM((_BPW,), jnp.int32),
        pltpu.VMEM((_BPW,), jnp.int32),
        pltpu.VMEM((_BPW, D), jnp.float32),
        pltpu.VMEM((_BPW, D), jnp.float32),
        pltpu.VMEM((_BPW,), jnp.float32),
        pltpu.VMEM((_BPW,), jnp.float32),
        pltpu.SemaphoreType.DMA,
    ],
    compiler_params=pltpu.CompilerParams(use_tc_tiling_on_sc=False),
)
def _sc_gather(uid_hbm, iid_hbm, uemb_hbm, iemb_hbm, ub_hbm, ib_hbm,
               urows_hbm, irows_hbm, ubg_hbm, ibg_hbm,
               uid_v, iid_v, urows_v, irows_v, ub_v, ib_v, sem):
    wid = lax.axis_index("s") * _NC + lax.axis_index("c")
    base = wid * _BPW
    sl = pl.ds(base, _BPW)
    pltpu.sync_copy(uid_hbm.at[sl], uid_v)
    pltpu.sync_copy(iid_hbm.at[sl], iid_v)
    pltpu.async_copy(uemb_hbm.at[uid_v], urows_v, sem).wait()
    pltpu.async_copy(iemb_hbm.at[iid_v], irows_v, sem).wait()
    pltpu.async_copy(ub_hbm.at[uid_v], ub_v, sem).wait()
    pltpu.async_copy(ib_hbm.at[iid_v], ib_v, sem).wait()
    pltpu.sync_copy(urows_v, urows_hbm.at[sl])
    pltpu.sync_copy(irows_v, irows_hbm.at[sl])
    pltpu.sync_copy(ub_v, ubg_hbm.at[sl])
    pltpu.sync_copy(ib_v, ibg_hbm.at[sl])


_RPB = 256


def _tc_body(ufull_ref, ifull_ref, ub_ref, ib_ref, gb_ref, out_ref,
             inter_ref):
    @pl.when(pl.program_id(0) == 0)
    def _():
        inter_ref[...] = jnp.sum(ufull_ref[...] * ifull_ref[...],
                                 axis=1)[None, :]
    out_ref[...] = ub_ref[...] + ib_ref[...] + inter_ref[...] + gb_ref[0]


def _tc_broadcast(urows, irows, ubg, ibg, global_bias):
    return pl.pallas_call(
        _tc_body,
        grid=(B // _RPB,),
        in_specs=[
            pl.BlockSpec((B, D), lambda i: (0, 0)),
            pl.BlockSpec((B, D), lambda i: (0, 0)),
            pl.BlockSpec((_RPB, 1), lambda i: (i, 0)),
            pl.BlockSpec((_RPB, 1), lambda i: (i, 0)),
            pl.BlockSpec(memory_space=pltpu.SMEM),
        ],
        out_specs=pl.BlockSpec((_RPB, B), lambda i: (i, 0)),
        out_shape=jax.ShapeDtypeStruct((B, B), jnp.float32),
        scratch_shapes=[pltpu.VMEM((1, B), jnp.float32)],
    )(urows, irows, ubg.reshape(B, 1), ibg.reshape(B, 1), global_bias)


def kernel(user_ids, item_ids, user_emb_table, item_emb_table,
           user_bias_table, item_bias_table, global_bias):
    uid = user_ids.astype(jnp.int32)
    iid = item_ids.astype(jnp.int32)
    urows, irows, ubg, ibg = _sc_gather(
        uid, iid, user_emb_table, item_emb_table,
        user_bias_table.reshape(-1), item_bias_table.reshape(-1))
    return _tc_broadcast(urows, irows, ubg, ibg, global_bias)


# speedup vs baseline: 1.4862x; 1.4862x over previous
import functools

import jax
import jax.numpy as jnp
from jax import lax
from jax.experimental import pallas as pl
from jax.experimental.pallas import tpu as pltpu
from jax.experimental.pallas import tpu_sc as plsc

B = 4096
D = 64

_info = plsc.get_sparse_core_info()
_NC = _info.num_cores
_NS = _info.num_subcores
_NW = _NC * _NS
_BPW = B // _NW
_CH = 16
_NCH = _BPW // _CH


@functools.partial(
    pl.kernel,
    mesh=plsc.VectorSubcoreMesh(core_axis_name="c", subcore_axis_name="s"),
    out_type=[
        jax.ShapeDtypeStruct((B, D), jnp.float32),
        jax.ShapeDtypeStruct((B, D), jnp.float32),
    ],
    scratch_types=[
        pltpu.VMEM((_BPW,), jnp.int32),
        pltpu.VMEM((_BPW,), jnp.int32),
        pltpu.VMEM((_BPW, D), jnp.float32),
        pltpu.VMEM((_BPW, D), jnp.float32),
        pltpu.SemaphoreType.DMA,
        pltpu.SemaphoreType.DMA,
    ],
    compiler_params=pltpu.CompilerParams(use_tc_tiling_on_sc=True),
)
def _sc_gather_rows(uid_hbm, iid_hbm, uemb_hbm, iemb_hbm,
                    urows_hbm, irows_hbm,
                    uid_v, iid_v, urows_v, irows_v, sem_u, sem_i):
    wid = lax.axis_index("s") * _NC + lax.axis_index("c")
    base = wid * _BPW
    sl = pl.ds(base, _BPW)
    pltpu.sync_copy(uid_hbm.at[sl], uid_v)
    pltpu.sync_copy(iid_hbm.at[sl], iid_v)

    def chunk(c, carry):
        uvec = uid_v[pl.ds(c * _CH, _CH)]
        ivec = iid_v[pl.ds(c * _CH, _CH)]
        descs = []
        for k in range(_CH):
            r = c * _CH + k
            dst = pl.ds(r, 1)
            descs.append(pltpu.async_copy(
                uemb_hbm.at[pl.ds(uvec[k], 1)], urows_v.at[dst], sem_u))
            descs.append(pltpu.async_copy(
                iemb_hbm.at[pl.ds(ivec[k], 1)], irows_v.at[dst], sem_i))
        for dsc in descs:
            dsc.wait()
        return carry

    lax.fori_loop(0, _NCH, chunk, 0)
    pltpu.sync_copy(urows_v, urows_hbm.at[sl])
    pltpu.sync_copy(irows_v, irows_hbm.at[sl])


@functools.partial(
    pl.kernel,
    mesh=plsc.VectorSubcoreMesh(core_axis_name="c", subcore_axis_name="s"),
    out_type=[
        jax.ShapeDtypeStruct((B,), jnp.float32),
        jax.ShapeDtypeStruct((B,), jnp.float32),
    ],
    scratch_types=[
        pltpu.VMEM((_BPW,), jnp.int32),
        pltpu.VMEM((_BPW,), jnp.int32),
        pltpu.VMEM((_BPW,), jnp.float32),
        pltpu.VMEM((_BPW,), jnp.float32),
        pltpu.SemaphoreType.DMA,
    ],
    compiler_params=pltpu.CompilerParams(use_tc_tiling_on_sc=False),
)
def _sc_gather_bias(uid_hbm, iid_hbm, ub_hbm, ib_hbm, ubg_hbm, ibg_hbm,
                    uid_v, iid_v, ub_v, ib_v, sem):
    wid = lax.axis_index("s") * _NC + lax.axis_index("c")
    base = wid * _BPW
    sl = pl.ds(base, _BPW)
    pltpu.sync_copy(uid_hbm.at[sl], uid_v)
    pltpu.sync_copy(iid_hbm.at[sl], iid_v)
    pltpu.async_copy(ub_hbm.at[uid_v], ub_v, sem).wait()
    pltpu.async_copy(ib_hbm.at[iid_v], ib_v, sem).wait()
    pltpu.sync_copy(ub_v, ubg_hbm.at[sl])
    pltpu.sync_copy(ib_v, ibg_hbm.at[sl])


_RPB = 256


def _tc_body(ufull_ref, ifull_ref, ub_ref, ib_ref, gb_ref, out_ref,
             inter_ref):
    @pl.when(pl.program_id(0) == 0)
    def _():
        inter_ref[...] = jnp.sum(ufull_ref[...] * ifull_ref[...],
                                 axis=1)[None, :]
    out_ref[...] = ub_ref[...] + ib_ref[...] + inter_ref[...] + gb_ref[0]


def _tc_broadcast(urows, irows, ubg, ibg, global_bias):
    return pl.pallas_call(
        _tc_body,
        grid=(B // _RPB,),
        in_specs=[
            pl.BlockSpec((B, D), lambda i: (0, 0)),
            pl.BlockSpec((B, D), lambda i: (0, 0)),
            pl.BlockSpec((_RPB, 1), lambda i: (i, 0)),
            pl.BlockSpec((_RPB, 1), lambda i: (i, 0)),
            pl.BlockSpec(memory_space=pltpu.SMEM),
        ],
        out_specs=pl.BlockSpec((_RPB, B), lambda i: (i, 0)),
        out_shape=jax.ShapeDtypeStruct((B, B), jnp.float32),
        scratch_shapes=[pltpu.VMEM((1, B), jnp.float32)],
    )(urows, irows, ubg.reshape(B, 1), ibg.reshape(B, 1), global_bias)


def kernel(user_ids, item_ids, user_emb_table, item_emb_table,
           user_bias_table, item_bias_table, global_bias):
    uid = user_ids.astype(jnp.int32)
    iid = item_ids.astype(jnp.int32)
    urows, irows = _sc_gather_rows(uid, iid, user_emb_table, item_emb_table)
    ubg, ibg = _sc_gather_bias(uid, iid, user_bias_table.reshape(-1),
                               item_bias_table.reshape(-1))
    return _tc_broadcast(urows, irows, ubg, ibg, global_bias)
